# baseline (speedup 1.0000x reference)
import functools

import jax
import jax.numpy as jnp
from jax import lax
from jax.experimental import pallas as pl
from jax.experimental.pallas import tpu as pltpu
from jax.experimental.pallas import tpu_sc as plsc

_CAP = 16384
_WB = 4096
_SB = 1024
_NW = 32
_BPW = _SB // _NW
_CHUNK = 8
_NCH = _BPW // _CHUNK


def _match_body(sample_ref, idx_ref, vidx_ref, d1_ref, d2_ref):
    i = pl.program_id(0)
    s = sample_ref[0]
    w = idx_ref[...]
    eq = w == s
    jio = lax.broadcasted_iota(jnp.int32, (_WB, 128), 0)
    lastj = jnp.max(jnp.where(eq, jio, -1), axis=0, keepdims=True)
    ow = lastj >= 0
    lanes = lax.broadcasted_iota(jnp.int32, (1, 128), 1)
    row = i * 128 + lanes
    pad = _SB + (row >> 5)
    vidx_ref[0] = jnp.where(ow, lastj, 0)
    d1_ref[0] = jnp.where(ow, pad, row)
    d2_ref[0] = jnp.where(ow, row, pad)


def _resolve_writes(sample3, idx2):
    return pl.pallas_call(
        _match_body,
        grid=(8,),
        in_specs=[
            pl.BlockSpec((1, 1, 128), lambda i: (i, 0, 0)),
            pl.BlockSpec((_WB, 1), lambda i: (0, 0)),
        ],
        out_specs=[
            pl.BlockSpec((1, 1, 128), lambda i: (i, 0, 0)),
            pl.BlockSpec((1, 1, 128), lambda i: (i, 0, 0)),
            pl.BlockSpec((1, 1, 128), lambda i: (i, 0, 0)),
        ],
        out_shape=[
            jax.ShapeDtypeStruct((8, 1, 128), jnp.int32),
            jax.ShapeDtypeStruct((8, 1, 128), jnp.int32),
            jax.ShapeDtypeStruct((8, 1, 128), jnp.int32),
        ],
    )(sample3, idx2)


def _sc_gather(mem, val, sample, vidx, d1, d2):
    @functools.partial(
        pl.kernel,
        mesh=plsc.VectorSubcoreMesh(core_axis_name="c", subcore_axis_name="s"),
        compiler_params=pltpu.CompilerParams(use_tc_tiling_on_sc=False),
        out_type=jax.ShapeDtypeStruct((_SB + _NW, 3, 32, 32), jnp.float32),
        scratch_types=[
            pltpu.VMEM((_BPW,), jnp.int32),
            pltpu.VMEM((_BPW,), jnp.int32),
            pltpu.VMEM((_NCH, _CHUNK), jnp.int32),
            pltpu.VMEM((_NCH, _CHUNK), jnp.int32),
            pltpu.VMEM((2, _CHUNK, 3, 32, 32), jnp.float32),
            pltpu.VMEM((2, _CHUNK, 3, 32, 32), jnp.float32),
            pltpu.SemaphoreType.DMA,
            pltpu.SemaphoreType.DMA((2,)),
            pltpu.SemaphoreType.DMA((2,)),
            pltpu.SemaphoreType.DMA((2,)),
            pltpu.SemaphoreType.DMA((2,)),
        ],
    )
    def k(mem_hbm, val_hbm, samp_hbm, vidx_hbm, d1_hbm, d2_hbm, out_hbm,
          sidx_v, vidx_v, d1_v, d2_v, bufa, bufb,
          semi, semga, semgb, semsa, semsb):
        wid = lax.axis_index("s") * 2 + lax.axis_index("c")
        base = wid * _BPW
        ci = [
            pltpu.make_async_copy(samp_hbm.at[pl.ds(base, _BPW)], sidx_v, semi),
            pltpu.make_async_copy(vidx_hbm.at[pl.ds(base, _BPW)], vidx_v, semi),
            pltpu.make_async_copy(d1_hbm.at[wid], d1_v, semi),
            pltpu.make_async_copy(d2_hbm.at[wid], d2_v, semi),
        ]
        for c in ci:
            c.start()
        for c in ci:
            c.wait()
        ga = [None] * _NCH
        gb = [None] * _NCH
        sa = [None] * _NCH
        sb = [None] * _NCH
        for c in range(_NCH + 1):
            if c >= 1:
                g = c - 1
                sg = g & 1
                ga[g].wait()
                sa[g] = pltpu.make_async_copy(
                    bufa.at[sg], out_hbm.at[d1_v.at[g]], semsa.at[sg])
                sa[g].start()
                gb[g].wait()
                sb[g] = pltpu.make_async_copy(
                    bufb.at[sg], out_hbm.at[d2_v.at[g]], semsb.at[sg])
                sb[g].start()
            if c < _NCH:
                s = c & 1
                if c >= 2:
                    sa[c - 2].wait()
                    sb[c - 2].wait()
                ga[c] = pltpu.make_async_copy(
                    mem_hbm.at[sidx_v.at[pl.ds(c * _CHUNK, _CHUNK)]],
                    bufa.at[s], semga.at[s])
                ga[c].start()
                gb[c] = pltpu.make_async_copy(
                    val_hbm.at[vidx_v.at[pl.ds(c * _CHUNK, _CHUNK)]],
                    bufb.at[s], semgb.at[s])
                gb[c].start()
        sa[_NCH - 2].wait()
        sb[_NCH - 2].wait()
        sa[_NCH - 1].wait()
        sb[_NCH - 1].wait()

    return k(mem, val, sample, vidx, d1, d2)


def kernel(mem, idx, val, sample_idx):
    sample3 = sample_idx.reshape(8, 1, 128)
    idx2 = idx.reshape(_WB, 1)
    vidx, d1, d2 = _resolve_writes(sample3, idx2)
    outp = _sc_gather(mem, val, sample_idx, vidx.reshape(_SB),
                      d1.reshape(_NW, _NCH, _CHUNK),
                      d2.reshape(_NW, _NCH, _CHUNK))
    return outp[:_SB]


# speedup vs baseline: 3.6119x; 3.6119x over previous
import functools

import jax
import jax.numpy as jnp
from jax import lax
from jax.experimental import pallas as pl
from jax.experimental.pallas import tpu as pltpu
from jax.experimental.pallas import tpu_sc as plsc

_CAP = 16384
_WB = 4096
_SB = 1024
_NW = 32
_BPW = _SB // _NW
_CHUNK = 8
_NCH = _BPW // _CHUNK


def _match_body(sample_ref, idx_ref, code_ref):
    s = sample_ref[0]
    w = idx_ref[...]
    eq = w == s
    jio = lax.broadcasted_iota(jnp.int32, (_WB, 128), 0)
    lastj = jnp.max(jnp.where(eq, jio, -1), axis=0, keepdims=True)
    code_ref[0] = jnp.where(lastj >= 0, _CAP + lastj, s)


def _resolve_writes(sample3, idx2):
    return pl.pallas_call(
        _match_body,
        grid=(8,),
        in_specs=[
            pl.BlockSpec((1, 1, 128), lambda i: (i, 0, 0)),
            pl.BlockSpec((_WB, 1), lambda i: (0, 0)),
        ],
        out_specs=pl.BlockSpec((1, 1, 128), lambda i: (i, 0, 0)),
        out_shape=jax.ShapeDtypeStruct((8, 1, 128), jnp.int32),
    )(sample3, idx2)


def _sc_gather(mem3, val3, code):
    @functools.partial(
        pl.kernel,
        mesh=plsc.VectorSubcoreMesh(core_axis_name="c", subcore_axis_name="s"),
        out_type=jax.ShapeDtypeStruct((_SB, 24, 128), jnp.float32),
        scratch_types=[
            pltpu.VMEM((_BPW,), jnp.int32),
            pltpu.VMEM((2, _CHUNK, 24, 128), jnp.float32),
            pltpu.SemaphoreType.DMA,
            pltpu.SemaphoreType.DMA((2,)),
            pltpu.SemaphoreType.DMA((2,)),
        ],
    )
    def k(mem_hbm, val_hbm, code_hbm, out_hbm, code_v, buf, semi, semg, semw):
        wid = lax.axis_index("s") * 2 + lax.axis_index("c")
        base = wid * _BPW
        pltpu.sync_copy(code_hbm.at[pl.ds(base, _BPW)], code_v)
        cw = [None] * _NCH
        for c in range(_NCH):
            s = c & 1
            if c >= 2:
                cw[c - 2].wait()
            fired = []
            for r in range(_CHUNK):
                i = c * _CHUNK + r
                vec = code_v[pl.ds((i // 16) * 16, 16)]
                scode = vec[i % 16]
                dst = buf.at[s, pl.ds(r, 1)]

                @pl.when(scode < _CAP)
                def _fetch_mem(dst=dst, scode=scode, s=s):
                    pltpu.make_async_copy(
                        mem_hbm.at[pl.ds(scode, 1)], dst, semg.at[s]).start()

                @pl.when(scode >= _CAP)
                def _fetch_val(dst=dst, scode=scode, s=s):
                    pltpu.make_async_copy(
                        val_hbm.at[pl.ds(scode - _CAP, 1)], dst,
                        semg.at[s]).start()

                fired.append(dst)
            for dst in fired:
                pltpu.make_async_copy(
                    mem_hbm.at[pl.ds(0, 1)], dst, semg.at[s]).wait()
            cw[c] = pltpu.make_async_copy(
                buf.at[s], out_hbm.at[pl.ds(base + c * _CHUNK, _CHUNK)],
                semw.at[s])
            cw[c].start()
        cw[_NCH - 2].wait()
        cw[_NCH - 1].wait()

    return k(mem3, val3, code)


def kernel(mem, idx, val, sample_idx):
    sample3 = sample_idx.reshape(8, 1, 128)
    idx2 = idx.reshape(_WB, 1)
    code = _resolve_writes(sample3, idx2)
    out3 = _sc_gather(mem.reshape(_CAP, 24, 128), val.reshape(_WB, 24, 128),
                      code.reshape(_SB))
    return out3.reshape(_SB, 3, 32, 32)


# speedup vs baseline: 3.6552x; 1.0120x over previous
import functools

import jax
import jax.numpy as jnp
from jax import lax
from jax.experimental import pallas as pl
from jax.experimental.pallas import tpu as pltpu
from jax.experimental.pallas import tpu_sc as plsc

_CAP = 16384
_WB = 4096
_SB = 1024
_NW = 32
_BPW = _SB // _NW
_CHUNK = 8
_NCH = _BPW // _CHUNK


def _match_body(sample_ref, idx_ref, code_ref):
    s = sample_ref[0]
    w = idx_ref[...]
    eq = w == s
    jio = lax.broadcasted_iota(jnp.int32, (_WB, 128), 0)
    lastj = jnp.max(jnp.where(eq, jio, -1), axis=0, keepdims=True)
    code_ref[0] = jnp.where(lastj >= 0, _CAP + lastj, s)


def _resolve_writes(sample3, idx2):
    return pl.pallas_call(
        _match_body,
        grid=(8,),
        in_specs=[
            pl.BlockSpec((1, 1, 128), lambda i: (i, 0, 0)),
            pl.BlockSpec((_WB, 1), lambda i: (0, 0)),
        ],
        out_specs=pl.BlockSpec((1, 1, 128), lambda i: (i, 0, 0)),
        out_shape=jax.ShapeDtypeStruct((8, 1, 128), jnp.int32),
    )(sample3, idx2)


def _sc_gather(mem3, val3, code):
    @functools.partial(
        pl.kernel,
        mesh=plsc.VectorSubcoreMesh(core_axis_name="c", subcore_axis_name="s"),
        out_type=jax.ShapeDtypeStruct((_SB, 24, 128), jnp.float32),
        scratch_types=[
            pltpu.VMEM((_BPW,), jnp.int32),
            pltpu.VMEM((_NCH, _CHUNK, 24, 128), jnp.float32),
            pltpu.SemaphoreType.DMA,
            pltpu.SemaphoreType.DMA((_NCH,)),
            pltpu.SemaphoreType.DMA((_NCH,)),
        ],
    )
    def k(mem_hbm, val_hbm, code_hbm, out_hbm, code_v, buf, semi, semg, semw):
        wid = lax.axis_index("s") * 2 + lax.axis_index("c")
        base = wid * _BPW
        pltpu.sync_copy(code_hbm.at[pl.ds(base, _BPW)], code_v)
        for c in range(_NCH):
            for r in range(_CHUNK):
                i = c * _CHUNK + r
                vec = code_v[pl.ds((i // 16) * 16, 16)]
                scode = vec[i % 16]
                dst = buf.at[c, pl.ds(r, 1)]

                @pl.when(scode < _CAP)
                def _fetch_mem(dst=dst, scode=scode, c=c):
                    pltpu.make_async_copy(
                        mem_hbm.at[pl.ds(scode, 1)], dst, semg.at[c]).start()

                @pl.when(scode >= _CAP)
                def _fetch_val(dst=dst, scode=scode, c=c):
                    pltpu.make_async_copy(
                        val_hbm.at[pl.ds(scode - _CAP, 1)], dst,
                        semg.at[c]).start()

        cw = [None] * _NCH
        for c in range(_NCH):
            for r in range(_CHUNK):
                pltpu.make_async_copy(
                    mem_hbm.at[pl.ds(0, 1)], buf.at[c, pl.ds(r, 1)],
                    semg.at[c]).wait()
            cw[c] = pltpu.make_async_copy(
                buf.at[c], out_hbm.at[pl.ds(base + c * _CHUNK, _CHUNK)],
                semw.at[c])
            cw[c].start()
        for c in range(_NCH):
            cw[c].wait()

    return k(mem3, val3, code)


def kernel(mem, idx, val, sample_idx):
    sample3 = sample_idx.reshape(8, 1, 128)
    idx2 = idx.reshape(_WB, 1)
    code = _resolve_writes(sample3, idx2)
    out3 = _sc_gather(mem.reshape(_CAP, 24, 128), val.reshape(_WB, 24, 128),
                      code.reshape(_SB))
    return out3.reshape(_SB, 3, 32, 32)
